# Initial kernel scaffold; baseline (speedup 1.0000x reference)
import functools

import jax
import jax.numpy as jnp
from jax import lax
from jax.experimental import pallas as pl
from jax.experimental.pallas import tpu as pltpu
from jax.experimental.pallas import tpu_sc as plsc

NC = 2
NS = 16
NW = NC * NS
L = 16

B = 80
DCHUNKS = 8
ACC_W = 144


def _sc_edge_pass(n_ent, d, e, ent_emb, rel_emb, src, dst, rid, zeros):
    rows_per_tile = n_ent // NS
    epw = e // NW
    nblk = epw // B

    mesh = plsc.VectorSubcoreMesh(core_axis_name="c", subcore_axis_name="s")

    @functools.partial(
        pl.kernel,
        mesh=mesh,
        out_type=jax.ShapeDtypeStruct((NC, n_ent, ACC_W), jnp.float32),
        scratch_types=[
            pltpu.VMEM((B,), jnp.int32),
            pltpu.VMEM((B,), jnp.int32),
            pltpu.VMEM((B,), jnp.int32),
            pltpu.VMEM((B, 128), jnp.float32),
            pltpu.VMEM((B, 128), jnp.float32),
            pltpu.VMEM((B, 128), jnp.float32),
            pltpu.VMEM((B, ACC_W), jnp.float32),
            pltpu.VMEM((B,), jnp.float32),
            pltpu.VMEM((B,), jnp.float32),
            pltpu.VMEM_SHARED((n_ent, ACC_W), jnp.float32),
            pltpu.SemaphoreType.DMA,
        ],
    )
    def sc_kernel(ent_hbm, rel_hbm, src_hbm, dst_hbm, rid_hbm, zeros_hbm,
                  part_hbm, sidx_v, didx_v, ridx_v, srow_v, rrow_v, drow_v,
                  msg_v, score_v, ex_v, acc_sh, sem):
        cid = lax.axis_index("c")
        sid = lax.axis_index("s")
        wid = sid * NC + cid

        pltpu.sync_copy(zeros_hbm, acc_sh.at[pl.ds(sid * rows_per_tile,
                                                   rows_per_tile)])

        @pl.loop(0, B)
        def _zero_pad(r):
            msg_v.at[pl.ds(r, 1), pl.ds(ACC_W - L, L)][...] = (
                jnp.zeros((1, L), jnp.float32))

        plsc.subcore_barrier()

        ebase = wid * epw

        @pl.loop(0, nblk)
        def _block(blk):
            base = ebase + blk * B
            pltpu.sync_copy(src_hbm.at[pl.ds(base, B)], sidx_v)
            pltpu.sync_copy(dst_hbm.at[pl.ds(base, B)], didx_v)
            pltpu.sync_copy(rid_hbm.at[pl.ds(base, B)], ridx_v)

            c1 = pltpu.async_copy(ent_hbm.at[sidx_v], srow_v, sem)
            c2 = pltpu.async_copy(rel_hbm.at[ridx_v], rrow_v, sem)
            c3 = pltpu.async_copy(ent_hbm.at[didx_v], drow_v, sem)
            c1.wait()
            c2.wait()
            c3.wait()

            @pl.loop(0, B)
            def _row(r):
                acc = jnp.zeros((1, L), jnp.float32)
                for c in range(DCHUNKS):
                    sl = (pl.ds(r, 1), pl.ds(c * L, L))
                    comp = srow_v.at[*sl][...] * rrow_v.at[*sl][...]
                    msg_v.at[*sl][...] = comp
                    acc += comp * drow_v.at[*sl][...]
                score_v[r] = jnp.sum(jnp.reshape(acc, (L,)))

            @pl.loop(0, B // L)
            def _exp(g):
                s = score_v.at[pl.ds(g * L, L)][...]
                ex_v.at[pl.ds(g * L, L)][...] = jnp.exp(jnp.minimum(s, 75.0))

            lane0 = lax.iota(jnp.int32, L) == 0

            @pl.loop(0, B)
            def _scale(r):
                ev = jnp.broadcast_to(ex_v[r], (1, L))
                for c in range(DCHUNKS):
                    sl = (pl.ds(r, 1), pl.ds(c * L, L))
                    msg_v.at[*sl][...] = msg_v.at[*sl][...] * ev
                msg_v.at[pl.ds(r, 1), pl.ds(128, L)][...] = jnp.where(
                    lane0, ev, jnp.zeros((1, L), jnp.float32))

            pltpu.sync_copy(msg_v, acc_sh.at[didx_v], add=True)

        plsc.subcore_barrier()
        pltpu.sync_copy(
            acc_sh.at[pl.ds(sid * rows_per_tile, rows_per_tile)],
            part_hbm.at[cid, pl.ds(sid * rows_per_tile, rows_per_tile)])

    return sc_kernel(ent_emb, rel_emb, src, dst, rid, zeros)


def _tc_finish_body(part_ref, w_ref, out_ref):
    p = part_ref[0] + part_ref[1]
    neigh = p[:, :128] / (p[:, 128:129] + 1e-16)
    out_ref[...] = jnp.tanh(
        jnp.dot(neigh, w_ref[...], preferred_element_type=jnp.float32))


def _tc_finish(part, neigh_w):
    n_ent = part.shape[1]
    return pl.pallas_call(
        _tc_finish_body,
        out_shape=jax.ShapeDtypeStruct((n_ent, 128), jnp.float32),
    )(part, neigh_w)


def kernel(ent_emb, rel_emb, edge_index, rel_id, neigh_w):
    n_ent, d = ent_emb.shape
    e = rel_id.shape[0]
    src = edge_index[0].astype(jnp.int32)
    dst = edge_index[1].astype(jnp.int32)
    rid = rel_id.astype(jnp.int32)
    zeros = jnp.zeros((n_ent // NS, ACC_W), jnp.float32)
    part = _sc_edge_pass(n_ent, d, e, ent_emb, rel_emb, src, dst, rid, zeros)
    return _tc_finish(part, neigh_w)


# speedup vs baseline: 6.6474x; 6.6474x over previous
import functools

import jax
import jax.numpy as jnp
from jax import lax
from jax.experimental import pallas as pl
from jax.experimental.pallas import tpu as pltpu
from jax.experimental.pallas import tpu_sc as plsc

NC = 2
NS = 16
NW = NC * NS
L = 16

B = 64
TB = 16
DCHUNKS = 8
ACC_W = 144


def _sc_edge_pass(n_ent, d, e, ent_emb, rel_emb, src, dst, rid, zeros):
    rows_per_tile = n_ent // NS
    epw = e // NW
    nblk = (epw - TB) // B
    assert nblk * B + TB == epw

    mesh = plsc.VectorSubcoreMesh(core_axis_name="c", subcore_axis_name="s")

    @functools.partial(
        pl.kernel,
        mesh=mesh,
        compiler_params=pltpu.CompilerParams(
            use_tc_tiling_on_sc=False, needs_layout_passes=False),
        out_type=jax.ShapeDtypeStruct((NC, n_ent, ACC_W), jnp.float32),
        scratch_types=[
            pltpu.VMEM((B,), jnp.int32),
            pltpu.VMEM((B,), jnp.int32),
            pltpu.VMEM((B,), jnp.int32),
            pltpu.VMEM((B, 128), jnp.float32),
            pltpu.VMEM((B, 128), jnp.float32),
            pltpu.VMEM((B, 128), jnp.float32),
            pltpu.VMEM((B, ACC_W), jnp.float32),
            pltpu.VMEM_SHARED((n_ent, ACC_W), jnp.float32),
            pltpu.SemaphoreType.DMA,
        ],
    )
    def sc_kernel(ent_hbm, rel_hbm, src_hbm, dst_hbm, rid_hbm, zeros_hbm,
                  part_hbm, sidx_v, didx_v, ridx_v, srow_v, rrow_v, drow_v,
                  msg_v, acc_sh, sem):
        cid = lax.axis_index("c")
        sid = lax.axis_index("s")
        wid = sid * NC + cid

        pltpu.sync_copy(zeros_hbm, acc_sh.at[pl.ds(sid * rows_per_tile,
                                                   rows_per_tile)])
        plsc.subcore_barrier()

        ebase = wid * epw
        lane0 = lax.iota(jnp.int32, L) == 0

        def do_block(base, nrows):
            pltpu.sync_copy(src_hbm.at[pl.ds(base, nrows)],
                            sidx_v.at[pl.ds(0, nrows)])
            pltpu.sync_copy(dst_hbm.at[pl.ds(base, nrows)],
                            didx_v.at[pl.ds(0, nrows)])
            pltpu.sync_copy(rid_hbm.at[pl.ds(base, nrows)],
                            ridx_v.at[pl.ds(0, nrows)])

            sl_rows = pl.ds(0, nrows)
            c1 = pltpu.async_copy(ent_hbm.at[sidx_v.at[sl_rows]],
                                  srow_v.at[sl_rows], sem)
            c2 = pltpu.async_copy(rel_hbm.at[ridx_v.at[sl_rows]],
                                  rrow_v.at[sl_rows], sem)
            c3 = pltpu.async_copy(ent_hbm.at[didx_v.at[sl_rows]],
                                  drow_v.at[sl_rows], sem)
            c1.wait()
            c2.wait()
            c3.wait()

            @pl.loop(0, nrows)
            def _row(r):
                acc = jnp.zeros((L,), jnp.float32)
                for c in range(DCHUNKS):
                    sl = pl.ds(c * L, L)
                    comp = srow_v[r, sl] * rrow_v[r, sl]
                    msg_v[r, sl] = comp
                    acc += comp * drow_v[r, sl]
                s = jnp.sum(acc)
                sv = jnp.broadcast_to(s, (L,))
                msg_v[r, pl.ds(128, L)] = jnp.exp(jnp.minimum(sv, 75.0))

            @pl.loop(0, nrows)
            def _scale(r):
                ev = msg_v[r, pl.ds(128, L)]
                for c in range(DCHUNKS):
                    sl = pl.ds(c * L, L)
                    msg_v[r, sl] = msg_v[r, sl] * ev
                msg_v[r, pl.ds(128, L)] = jnp.where(
                    lane0, ev, jnp.zeros((L,), jnp.float32))

            pltpu.sync_copy(msg_v.at[sl_rows],
                            acc_sh.at[didx_v.at[sl_rows]], add=True)

        @pl.loop(0, nblk)
        def _block(blk):
            do_block(ebase + blk * B, B)

        do_block(ebase + nblk * B, TB)

        plsc.subcore_barrier()
        pltpu.sync_copy(
            acc_sh.at[pl.ds(sid * rows_per_tile, rows_per_tile)],
            part_hbm.at[cid, pl.ds(sid * rows_per_tile, rows_per_tile)])

    return sc_kernel(ent_emb, rel_emb, src, dst, rid, zeros)


def _tc_finish_body(part_ref, w_ref, out_ref):
    p = part_ref[0] + part_ref[1]
    neigh = p[:, :128] / (p[:, 128:129] + 1e-16)
    out_ref[...] = jnp.tanh(
        jnp.dot(neigh, w_ref[...], preferred_element_type=jnp.float32))


def _tc_finish(part, neigh_w):
    n_ent = part.shape[1]
    return pl.pallas_call(
        _tc_finish_body,
        out_shape=jax.ShapeDtypeStruct((n_ent, 128), jnp.float32),
    )(part, neigh_w)


def kernel(ent_emb, rel_emb, edge_index, rel_id, neigh_w):
    n_ent, d = ent_emb.shape
    e = rel_id.shape[0]
    src = edge_index[0].astype(jnp.int32)
    dst = edge_index[1].astype(jnp.int32)
    rid = rel_id.astype(jnp.int32)
    zeros = jnp.zeros((n_ent // NS, ACC_W), jnp.float32)
    part = _sc_edge_pass(n_ent, d, e, ent_emb, rel_emb, src, dst, rid, zeros)
    return _tc_finish(part, neigh_w)
